# baseline (speedup 1.0000x reference)
import functools

import jax
import jax.numpy as jnp
from jax import lax
from jax.experimental import pallas as pl
from jax.experimental.pallas import tpu as pltpu
from jax.experimental.pallas import tpu_sc as plsc

N = 10000
E = 320000
D = 128
F = 64
NCLS = 10
NPAD = 10240
PADI = 10000
NCORE = 2
NSUB = 16
NW = NCORE * NSUB
CHUNK = 128
NCHUNK = 80
EPAD = NW * CHUNK * NCHUNK
RPT = NPAD // NSUB
F2 = F // 2
BLK = 256
NB = NPAD // BLK
HI = lax.Precision.HIGHEST
f32 = jnp.float32

_MESH = plsc.VectorSubcoreMesh(core_axis_name="c", subcore_axis_name="s")


@functools.partial(
    pl.kernel,
    out_type=jax.ShapeDtypeStruct((NCORE, 2, NPAD, F2), f32),
    mesh=_MESH,
    compiler_params=pltpu.CompilerParams(use_tc_tiling_on_sc=False),
    scratch_types=[
        pltpu.VMEM((NCHUNK, CHUNK), jnp.int32),
        pltpu.VMEM((NCHUNK, CHUNK), jnp.int32),
        pltpu.VMEM((2, 8 * CHUNK, F2), f32),
        pltpu.VMEM((64, F2), f32),
        pltpu.VMEM_SHARED((NPAD, F2), f32),
        pltpu.VMEM_SHARED((NPAD, F2), f32),
        pltpu.SemaphoreType.DMA,
        pltpu.SemaphoreType.DMA,
    ],
)
def _sc_edge_pass(tab0, tab1, gidx, sidx, out, gi, si, buf, zbuf, src, acc,
                  sem_g, sem_s):
    cid = lax.axis_index("c")
    sid = lax.axis_index("s")
    wid = cid * NSUB + sid
    base = sid * RPT
    NG = NCHUNK // 8
    pltpu.sync_copy(gidx.at[wid], gi)
    pltpu.sync_copy(sidx.at[wid], si)
    zv = jnp.zeros((16,), f32)
    for r in range(64):
        for c in range(F2 // 16):
            zbuf[r, pl.ds(c * 16, 16)] = zv

    for half, tab in enumerate((tab0, tab1)):
        def fire_g(g, x, tab=tab):
            for b in range(8):
                pltpu.async_copy(src.at[gi.at[8 * g + b]],
                                 buf.at[x, pl.ds(b * CHUNK, CHUNK)], sem_g)

        def fire_s(g, x, tab=tab):
            for b in range(8):
                pltpu.async_copy(buf.at[x, pl.ds(b * CHUNK, CHUNK)],
                                 acc.at[si.at[8 * g + b]], sem_s, add=True)

        def drain(sem, x, tab=tab):
            pltpu.make_async_copy(tab.at[pl.ds(0, 8 * CHUNK)], buf.at[x],
                                  sem).wait()

        pltpu.sync_copy(tab.at[pl.ds(base, RPT)], src.at[pl.ds(base, RPT)])

        @pl.loop(0, RPT // 64)
        def _zero(k):
            pltpu.sync_copy(zbuf, acc.at[pl.ds(base + k * 64, 64)])

        plsc.subcore_barrier()

        fire_g(0, 0)
        drain(sem_g, 0)
        fire_s(0, 0)
        fire_g(1, 1)

        @pl.loop(0, NG // 2 - 1)
        def _steady(t):
            g1 = 2 * t + 1
            drain(sem_g, 1)
            fire_s(g1, 1)
            drain(sem_s, 0)
            fire_g(g1 + 1, 0)
            drain(sem_g, 0)
            fire_s(g1 + 1, 0)
            drain(sem_s, 1)
            fire_g(g1 + 2, 1)

        g_last = NG - 1
        drain(sem_g, 1)
        fire_s(g_last, 1)
        drain(sem_s, 0)
        drain(sem_s, 1)

        plsc.subcore_barrier()
        pltpu.sync_copy(acc.at[pl.ds(base, RPT)],
                        out.at[cid, half, pl.ds(base, RPT)])


@functools.partial(
    pl.kernel,
    out_type=jax.ShapeDtypeStruct((3, NCORE, NPAD), f32),
    mesh=_MESH,
    compiler_params=pltpu.CompilerParams(use_tc_tiling_on_sc=False),
    scratch_types=[
        pltpu.VMEM((NCHUNK, CHUNK), jnp.int32),
        pltpu.VMEM((CHUNK,), f32),
        pltpu.VMEM_SHARED((NPAD,), f32),
        pltpu.VMEM_SHARED((NPAD,), f32),
        pltpu.VMEM_SHARED((NPAD,), f32),
        pltpu.SemaphoreType.DMA,
    ],
)
def _sc_degrees(vix, eix, dix, zer3, out, iv, ones, t0, t1, t2, sem):
    cid = lax.axis_index("c")
    sid = lax.axis_index("s")
    wid = cid * NSUB + sid
    for c in range(CHUNK // 16):
        ones[pl.ds(c * 16, 16)] = jnp.ones((16,), f32)
    base = sid * RPT
    for k, t in enumerate((t0, t1, t2)):
        pltpu.sync_copy(zer3.at[k, pl.ds(base, RPT)], t.at[pl.ds(base, RPT)])
    plsc.subcore_barrier()
    for slab, t in ((vix, t0), (eix, t1), (dix, t2)):
        pltpu.sync_copy(slab.at[wid], iv)

        @pl.loop(0, NCHUNK // 8)
        def _s(k8, slab=slab, t=t):
            for b in range(8):
                pltpu.async_copy(ones, t.at[iv.at[8 * k8 + b]], sem, add=True)
            pltpu.make_async_copy(slab.at[wid, pl.ds(0, 8)],
                                  iv.at[pl.ds(0, 8)], sem).wait()
    plsc.subcore_barrier()
    for k, t in enumerate((t0, t1, t2)):
        pltpu.sync_copy(t.at[pl.ds(base, RPT)], out.at[k, cid, pl.ds(base, RPT)])


def _scales(dblk, row0):
    rows = dblk.shape[0]
    rid = row0 + lax.broadcasted_iota(jnp.int32, (rows, 1), 0)
    m = (rid < N).astype(f32)
    dv = dblk[:, 0:1] + dblk[:, 1:2]
    de = dblk[:, 2:3] + dblk[:, 3:4]
    dg = dblk[:, 4:5] + dblk[:, 5:6]
    dvi = jnp.where(dv > 0, 1.0, 0.0) * lax.rsqrt(jnp.maximum(dv, 1.0)) * m
    dei = jnp.where(de > 0, 1.0, 0.0) / jnp.maximum(de, 1.0) * m
    dinv = lax.rsqrt(1.0 + dg) * m
    return dvi, dei, dinv


def _combine_weights(W_h1, W_h2, b_h1, W_g1, W_g2, b_g1):
    def body(wh1, wh2, bh1, wg1, wg2, bg1, w12h, bh, w12g, bg):
        w12h[...] = jnp.dot(wh1[...], wh2[...], precision=HI)
        bh[...] = jnp.dot(bh1[...], wh2[...], precision=HI)
        w12g[...] = jnp.dot(wg1[...], wg2[...], precision=HI)
        bg[...] = jnp.dot(bg1[...], wg2[...], precision=HI)

    return pl.pallas_call(
        body,
        out_shape=[
            jax.ShapeDtypeStruct((D, F), f32),
            jax.ShapeDtypeStruct((1, F), f32),
            jax.ShapeDtypeStruct((D, F), f32),
            jax.ShapeDtypeStruct((1, F), f32),
        ],
    )(W_h1, W_h2, b_h1.reshape(1, -1), W_g1, W_g2, b_g1.reshape(1, -1))


def _project(XH, XG, w12h, bh, w12g, bg, degt):
    def body(xh, xg, wh, bh_, wg, bg_, dg, mh0, mh1, mg0, mg1):
        i = pl.program_id(0)
        dvi, _, dinv = _scales(dg[...], i * BLK)
        mh = (jnp.dot(xh[...], wh[...], precision=HI) + bh_[...]) * dvi
        mg = (jnp.dot(xg[...], wg[...], precision=HI) + bg_[...]) * dinv
        mh0[...] = mh[:, :F2]
        mh1[...] = mh[:, F2:]
        mg0[...] = mg[:, :F2]
        mg1[...] = mg[:, F2:]

    return pl.pallas_call(
        body,
        grid=(NB,),
        in_specs=[
            pl.BlockSpec((BLK, D), lambda i: (i, 0)),
            pl.BlockSpec((BLK, D), lambda i: (i, 0)),
            pl.BlockSpec((D, F), lambda i: (0, 0)),
            pl.BlockSpec((1, F), lambda i: (0, 0)),
            pl.BlockSpec((D, F), lambda i: (0, 0)),
            pl.BlockSpec((1, F), lambda i: (0, 0)),
            pl.BlockSpec((BLK, 6), lambda i: (i, 0)),
        ],
        out_specs=[pl.BlockSpec((BLK, F2), lambda i: (i, 0))] * 4,
        out_shape=[jax.ShapeDtypeStruct((NPAD, F2), f32)] * 4,
    )(XH, XG, w12h, bh, w12g, bg, degt)


def _merge(p, degt, sel, add=None, bias=None):
    has_add = add is not None
    has_bias = bias is not None

    def body(*refs):
        i = pl.program_id(0)
        it = iter(refs)
        pr = next(it)[...]
        dg = next(it)[...]
        a0 = next(it)[...] if has_add else 0.0
        a1 = next(it)[...] if has_add else 0.0
        b = next(it)[...] if has_bias else None
        o0 = next(it)
        o1 = next(it)
        dvi, dei, dinv = _scales(dg, i * BLK)
        s = {"dvi": dvi, "dei": dei, "dinv": dinv}[sel]
        v0 = (pr[0, 0] + pr[1, 0] + a0) * s
        v1 = (pr[0, 1] + pr[1, 1] + a1) * s
        if has_bias:
            v0 = (v0 + b[:, :F2]) * s
            v1 = (v1 + b[:, F2:]) * s
        o0[...] = v0
        o1[...] = v1

    in_specs = [
        pl.BlockSpec((NCORE, 2, BLK, F2), lambda i: (0, 0, i, 0)),
        pl.BlockSpec((BLK, 6), lambda i: (i, 0)),
    ]
    args = [p, degt]
    if has_add:
        in_specs.append(pl.BlockSpec((BLK, F2), lambda i: (i, 0)))
        in_specs.append(pl.BlockSpec((BLK, F2), lambda i: (i, 0)))
        args.extend(add)
    if has_bias:
        in_specs.append(pl.BlockSpec((1, F), lambda i: (0, 0)))
        args.append(bias.reshape(1, -1))
    return pl.pallas_call(
        body,
        grid=(NB,),
        in_specs=in_specs,
        out_specs=[pl.BlockSpec((BLK, F2), lambda i: (i, 0))] * 2,
        out_shape=[jax.ShapeDtypeStruct((NPAD, F2), f32)] * 2,
    )(*args)


ABLK = 128
NAB = NPAD // ABLK


def _attn(h, g, Wa, ba, Wb, bb, Wc, bc):
    def body(hr0, hr1, gr0, gr1, wa, ba_, wb, bb_, wc, bc_,
             sh_o, fh_o, mzh_o, fg_o, mh, zh, fh, mg_, zg, fg):
        i = pl.program_id(0)

        @pl.when(i == 0)
        def _init():
            mh[...] = jnp.full((1, 1), -1e30, f32)
            zh[...] = jnp.zeros((1, 1), f32)
            fh[...] = jnp.zeros((1, F), f32)
            mg_[...] = jnp.full((1, 1), -1e30, f32)
            zg[...] = jnp.zeros((1, 1), f32)
            fg[...] = jnp.zeros((1, F), f32)

        rid = i * ABLK + lax.broadcasted_iota(jnp.int32, (ABLK, 1), 0)
        mask = rid < N

        def branch(x, m_ref, z_ref, f_ref, s_out):
            a = jnp.tanh(jnp.dot(x, wa[...], precision=HI) + ba_[...])
            bg = jnp.dot(x, wb[...], precision=HI) + bb_[...]
            bg = 1.0 / (1.0 + jnp.exp(-bg))
            s = jnp.dot(a * bg, wc[...], precision=HI) + bc_[...]
            s = jnp.where(mask, s, -1e30)
            if s_out is not None:
                s_out[...] = s
            m_old = m_ref[0, 0]
            z_old = z_ref[0, 0]
            m_new = jnp.maximum(m_old, jnp.max(s))
            corr = jnp.exp(m_old - m_new)
            e = jnp.exp(s - m_new)
            z_new = z_old * corr + jnp.sum(e)
            f_new = f_ref[...] * corr + jnp.sum(e * x, axis=0, keepdims=True)
            m_ref[...] = jnp.full((1, 1), m_new, f32)
            z_ref[...] = jnp.full((1, 1), z_new, f32)
            f_ref[...] = f_new
            return m_new, z_new, f_new

        mhv, zhv, fhv = branch(
            jnp.concatenate([hr0[...], hr1[...]], axis=1), mh, zh, fh, sh_o)
        _, zgv, fgv = branch(
            jnp.concatenate([gr0[...], gr1[...]], axis=1), mg_, zg, fg, None)
        fh_o[...] = fhv / zhv
        mzh_o[...] = jnp.concatenate(
            [jnp.full((1, 1), mhv, f32), jnp.full((1, 1), zhv, f32)], axis=1)
        fg_o[...] = fgv / zgv

    return pl.pallas_call(
        body,
        grid=(NAB,),
        in_specs=[
            pl.BlockSpec((ABLK, F2), lambda i: (i, 0)),
            pl.BlockSpec((ABLK, F2), lambda i: (i, 0)),
            pl.BlockSpec((ABLK, F2), lambda i: (i, 0)),
            pl.BlockSpec((ABLK, F2), lambda i: (i, 0)),
            pl.BlockSpec((F, 256), lambda i: (0, 0)),
            pl.BlockSpec((1, 256), lambda i: (0, 0)),
            pl.BlockSpec((F, 256), lambda i: (0, 0)),
            pl.BlockSpec((1, 256), lambda i: (0, 0)),
            pl.BlockSpec((256, 1), lambda i: (0, 0)),
            pl.BlockSpec((1, 1), lambda i: (0, 0)),
        ],
        out_specs=[
            pl.BlockSpec((ABLK, 1), lambda i: (i, 0)),
            pl.BlockSpec((1, F), lambda i: (0, 0)),
            pl.BlockSpec((1, 2), lambda i: (0, 0)),
            pl.BlockSpec((1, F), lambda i: (0, 0)),
        ],
        out_shape=[
            jax.ShapeDtypeStruct((NPAD, 1), f32),
            jax.ShapeDtypeStruct((1, F), f32),
            jax.ShapeDtypeStruct((1, 2), f32),
            jax.ShapeDtypeStruct((1, F), f32),
        ],
        scratch_shapes=[pltpu.VMEM((1, 1), f32), pltpu.VMEM((1, 1), f32),
                        pltpu.VMEM((1, F), f32), pltpu.VMEM((1, 1), f32),
                        pltpu.VMEM((1, 1), f32), pltpu.VMEM((1, F), f32)],
    )(h[0], h[1], g[0], g[1], Wa, ba.reshape(1, -1), Wb, bb.reshape(1, -1),
      Wc, bc.reshape(1, -1))


def _head(s_h, mz, feat_h, feat_g, Wo, bo, l1g, l1b, l2g, l2b, Wf, bf):
    def body(s, mzr, fh, fg, wo, bo_, g1, b1, g2, b2, wf, bf_, lo, ws):
        m = mzr[0, 0]
        z = mzr[0, 1]
        ws[...] = jnp.exp(s[...] - m) / z

        def ln(x, gg, bb_):
            mu = jnp.mean(x, axis=-1, keepdims=True)
            va = jnp.mean((x - mu) ** 2, axis=-1, keepdims=True)
            return (x - mu) * lax.rsqrt(va + 1e-5) * gg + bb_

        ha = ln(jnp.dot(fh[...], wo[...], precision=HI) + bo_[...], g1[...], b1[...])
        ga = ln(jnp.dot(fg[...], wo[...], precision=HI) + bo_[...], g1[...], b1[...])
        xc = ln(jnp.concatenate([ha, ga], axis=1), g2[...], b2[...])
        lo[...] = jnp.dot(xc, wf[...], precision=HI) + bf_[...]

    return pl.pallas_call(
        body,
        grid=(NB,),
        in_specs=[
            pl.BlockSpec((BLK, 1), lambda i: (i, 0)),
            pl.BlockSpec((1, 2), lambda i: (0, 0)),
            pl.BlockSpec((1, F), lambda i: (0, 0)),
            pl.BlockSpec((1, F), lambda i: (0, 0)),
            pl.BlockSpec((F, F), lambda i: (0, 0)),
            pl.BlockSpec((1, F), lambda i: (0, 0)),
            pl.BlockSpec((1, F), lambda i: (0, 0)),
            pl.BlockSpec((1, F), lambda i: (0, 0)),
            pl.BlockSpec((1, 2 * F), lambda i: (0, 0)),
            pl.BlockSpec((1, 2 * F), lambda i: (0, 0)),
            pl.BlockSpec((2 * F, NCLS), lambda i: (0, 0)),
            pl.BlockSpec((1, NCLS), lambda i: (0, 0)),
        ],
        out_specs=[
            pl.BlockSpec((1, NCLS), lambda i: (0, 0)),
            pl.BlockSpec((BLK, 1), lambda i: (i, 0)),
        ],
        out_shape=[
            jax.ShapeDtypeStruct((1, NCLS), f32),
            jax.ShapeDtypeStruct((NPAD, 1), f32),
        ],
    )(s_h, mz, feat_h, feat_g, Wo, bo.reshape(1, -1), l1g.reshape(1, -1),
      l1b.reshape(1, -1), l2g.reshape(1, -1), l2b.reshape(1, -1), Wf,
      bf.reshape(1, -1))


def kernel(X_H, X_G, hg_pairs, g_edge_index, W_h1, b_h1, W_h2, b_h2,
           W_g1, b_g1, W_g2, b_g2, Wa, ba, Wb, bb, Wc, bc, Wo, bo,
           ln1_g, ln1_b, ln2_g, ln2_b, Wf, bf):
    def slab(ix):
        pad = jnp.full((EPAD - E,), PADI, jnp.int32)
        return jnp.concatenate([ix, pad]).reshape(NW, NCHUNK, CHUNK)

    v_s = slab(hg_pairs[0])
    e_s = slab(hg_pairs[1])
    src_s = slab(g_edge_index[0])
    dst_s = slab(g_edge_index[1])
    zrows = jnp.zeros((NPAD - N, D), f32)
    XHp = jnp.concatenate([X_H, zrows], axis=0)
    XGp = jnp.concatenate([X_G, zrows], axis=0)

    zer3 = jnp.zeros((3, NPAD), f32)
    deg = _sc_degrees(v_s, e_s, dst_s, zer3)
    degt = jnp.transpose(deg.reshape(6, NPAD))
    w12h, bh, w12g, bg = _combine_weights(W_h1, W_h2, b_h1, W_g1, W_g2, b_g1)
    mh0, mh1, mg0, mg1 = _project(XHp, XGp, w12h, bh, w12g, bg, degt)
    Mh = (mh0, mh1)
    Mg = (mg0, mg1)

    p = _sc_edge_pass(Mh[0], Mh[1], v_s, e_s)
    q = _sc_edge_pass(Mg[0], Mg[1], src_s, dst_s)
    xe = _merge(p, degt, "dei")
    in2g = _merge(q, degt, "dinv", add=Mg, bias=b_g2)
    p = _sc_edge_pass(xe[0], xe[1], e_s, v_s)
    q = _sc_edge_pass(in2g[0], in2g[1], src_s, dst_s)
    in2 = _merge(p, degt, "dvi", bias=b_h2)
    g = _merge(q, degt, "dinv", add=in2g)
    p = _sc_edge_pass(in2[0], in2[1], v_s, e_s)
    xe2 = _merge(p, degt, "dei")
    p = _sc_edge_pass(xe2[0], xe2[1], e_s, v_s)
    h = _merge(p, degt, "dvi")

    s_h, feat_h, mz, feat_g = _attn(h, g, Wa, ba, Wb, bb, Wc, bc)
    logits, ws = _head(s_h, mz, feat_h, feat_g, Wo, bo,
                       ln1_g, ln1_b, ln2_g, ln2_b, Wf, bf)
    return logits, ws[:N, 0]


# speedup vs baseline: 1.0677x; 1.0677x over previous
import functools

import jax
import jax.numpy as jnp
from jax import lax
from jax.experimental import pallas as pl
from jax.experimental.pallas import tpu as pltpu
from jax.experimental.pallas import tpu_sc as plsc

N = 10000
E = 320000
D = 128
F = 64
NCLS = 10
NPAD = 10240
PADI = 10000
NCORE = 2
NSUB = 16
NW = NCORE * NSUB
CHUNK = 128
NCHUNK = 80
EPAD = NW * CHUNK * NCHUNK
RPT = NPAD // NSUB
F2 = F // 2
BLK = 256
NB = NPAD // BLK
HI = lax.Precision.HIGHEST
f32 = jnp.float32

_MESH = plsc.VectorSubcoreMesh(core_axis_name="c", subcore_axis_name="s")


@functools.partial(
    pl.kernel,
    out_type=jax.ShapeDtypeStruct((NCORE, 2, NPAD, F2), f32),
    mesh=_MESH,
    compiler_params=pltpu.CompilerParams(use_tc_tiling_on_sc=False),
    scratch_types=[
        pltpu.VMEM((NCHUNK, CHUNK), jnp.int32),
        pltpu.VMEM((NCHUNK, CHUNK), jnp.int32),
        pltpu.VMEM((4, CHUNK, F2), f32),
        pltpu.VMEM((64, F2), f32),
        pltpu.VMEM_SHARED((NPAD, F2), f32),
        pltpu.VMEM_SHARED((NPAD, F2), f32),
        pltpu.SemaphoreType.DMA,
        pltpu.SemaphoreType.DMA,
    ],
)
def _sc_edge_pass(tab0, tab1, gidx, sidx, out, gi, si, buf, zbuf, src, acc,
                  sem_g, sem_s):
    cid = lax.axis_index("c")
    sid = lax.axis_index("s")
    wid = cid * NSUB + sid
    base = sid * RPT
    pltpu.sync_copy(gidx.at[wid], gi)
    pltpu.sync_copy(sidx.at[wid], si)
    zv = jnp.zeros((16,), f32)
    for r in range(64):
        for c in range(F2 // 16):
            zbuf[r, pl.ds(c * 16, 16)] = zv

    def g_start(j, b):
        pltpu.async_copy(src.at[gi.at[j]], buf.at[b], sem_g)

    def g_wait(j, b):
        pltpu.make_async_copy(src.at[gi.at[j]], buf.at[b], sem_g).wait()

    def s_start(j, b):
        pltpu.async_copy(buf.at[b], acc.at[si.at[j]], sem_s, add=True)

    def s_wait(j, b):
        pltpu.make_async_copy(buf.at[b], acc.at[si.at[j]], sem_s).wait()

    for half, tab in enumerate((tab0, tab1)):
        pltpu.sync_copy(tab.at[pl.ds(base, RPT)], src.at[pl.ds(base, RPT)])

        @pl.loop(0, RPT // 64)
        def _zero(k):
            pltpu.sync_copy(zbuf, acc.at[pl.ds(base + k * 64, 64)])

        plsc.subcore_barrier()
        for b in range(3):
            g_start(b, b)

        g_wait(0, 0)
        s_start(0, 0)
        g_start(3, 3)
        for b in range(1, 4):
            g_wait(b, b)
            s_start(b, b)
            s_wait(b - 1, b - 1)
            g_start(b + 3, (b + 3) % 4)

        @pl.loop(1, NCHUNK // 4 - 1)
        def _steady(k):
            j0 = 4 * k
            for b in range(4):
                j = j0 + b
                g_wait(j, b)
                s_start(j, b)
                s_wait(j - 1, (b - 1) % 4)
                g_start(j + 3, (b + 3) % 4)

        jt = NCHUNK - 4
        g_wait(jt, 0)
        s_start(jt, 0)
        s_wait(jt - 1, 3)
        g_start(NCHUNK - 1, 3)
        for b in range(1, 4):
            g_wait(jt + b, b)
            s_start(jt + b, b)
            s_wait(jt + b - 1, b - 1)
        s_wait(NCHUNK - 1, 3)

        plsc.subcore_barrier()
        pltpu.sync_copy(acc.at[pl.ds(base, RPT)],
                        out.at[cid, half, pl.ds(base, RPT)])


@functools.partial(
    pl.kernel,
    out_type=jax.ShapeDtypeStruct((3, NCORE, NPAD), f32),
    mesh=_MESH,
    compiler_params=pltpu.CompilerParams(use_tc_tiling_on_sc=False),
    scratch_types=[
        pltpu.VMEM((NCHUNK, CHUNK), jnp.int32),
        pltpu.VMEM((CHUNK,), f32),
        pltpu.VMEM_SHARED((NPAD,), f32),
        pltpu.VMEM_SHARED((NPAD,), f32),
        pltpu.VMEM_SHARED((NPAD,), f32),
        pltpu.SemaphoreType.DMA,
    ],
)
def _sc_degrees(vix, eix, dix, zer3, out, iv, ones, t0, t1, t2, sem):
    cid = lax.axis_index("c")
    sid = lax.axis_index("s")
    wid = cid * NSUB + sid
    for c in range(CHUNK // 16):
        ones[pl.ds(c * 16, 16)] = jnp.ones((16,), f32)
    base = sid * RPT
    for k, t in enumerate((t0, t1, t2)):
        pltpu.sync_copy(zer3.at[k, pl.ds(base, RPT)], t.at[pl.ds(base, RPT)])
    plsc.subcore_barrier()
    for slab, t in ((vix, t0), (eix, t1), (dix, t2)):
        pltpu.sync_copy(slab.at[wid], iv)

        @pl.loop(0, NCHUNK // 8)
        def _s(k8, slab=slab, t=t):
            for b in range(8):
                pltpu.async_copy(ones, t.at[iv.at[8 * k8 + b]], sem, add=True)
            pltpu.make_async_copy(slab.at[wid, pl.ds(0, 8)],
                                  iv.at[pl.ds(0, 8)], sem).wait()
    plsc.subcore_barrier()
    for k, t in enumerate((t0, t1, t2)):
        pltpu.sync_copy(t.at[pl.ds(base, RPT)], out.at[k, cid, pl.ds(base, RPT)])


def _scales(dblk, row0):
    rows = dblk.shape[0]
    rid = row0 + lax.broadcasted_iota(jnp.int32, (rows, 1), 0)
    m = (rid < N).astype(f32)
    dv = dblk[:, 0:1] + dblk[:, 1:2]
    de = dblk[:, 2:3] + dblk[:, 3:4]
    dg = dblk[:, 4:5] + dblk[:, 5:6]
    dvi = jnp.where(dv > 0, 1.0, 0.0) * lax.rsqrt(jnp.maximum(dv, 1.0)) * m
    dei = jnp.where(de > 0, 1.0, 0.0) / jnp.maximum(de, 1.0) * m
    dinv = lax.rsqrt(1.0 + dg) * m
    return dvi, dei, dinv


def _combine_weights(W_h1, W_h2, b_h1, W_g1, W_g2, b_g1):
    def body(wh1, wh2, bh1, wg1, wg2, bg1, w12h, bh, w12g, bg):
        w12h[...] = jnp.dot(wh1[...], wh2[...], precision=HI)
        bh[...] = jnp.dot(bh1[...], wh2[...], precision=HI)
        w12g[...] = jnp.dot(wg1[...], wg2[...], precision=HI)
        bg[...] = jnp.dot(bg1[...], wg2[...], precision=HI)

    return pl.pallas_call(
        body,
        out_shape=[
            jax.ShapeDtypeStruct((D, F), f32),
            jax.ShapeDtypeStruct((1, F), f32),
            jax.ShapeDtypeStruct((D, F), f32),
            jax.ShapeDtypeStruct((1, F), f32),
        ],
    )(W_h1, W_h2, b_h1.reshape(1, -1), W_g1, W_g2, b_g1.reshape(1, -1))


def _project(XH, XG, w12h, bh, w12g, bg, degt):
    def body(xh, xg, wh, bh_, wg, bg_, dg, mh0, mh1, mg0, mg1):
        i = pl.program_id(0)
        dvi, _, dinv = _scales(dg[...], i * BLK)
        mh = (jnp.dot(xh[...], wh[...], precision=HI) + bh_[...]) * dvi
        mg = (jnp.dot(xg[...], wg[...], precision=HI) + bg_[...]) * dinv
        mh0[...] = mh[:, :F2]
        mh1[...] = mh[:, F2:]
        mg0[...] = mg[:, :F2]
        mg1[...] = mg[:, F2:]

    return pl.pallas_call(
        body,
        grid=(NB,),
        in_specs=[
            pl.BlockSpec((BLK, D), lambda i: (i, 0)),
            pl.BlockSpec((BLK, D), lambda i: (i, 0)),
            pl.BlockSpec((D, F), lambda i: (0, 0)),
            pl.BlockSpec((1, F), lambda i: (0, 0)),
            pl.BlockSpec((D, F), lambda i: (0, 0)),
            pl.BlockSpec((1, F), lambda i: (0, 0)),
            pl.BlockSpec((BLK, 6), lambda i: (i, 0)),
        ],
        out_specs=[pl.BlockSpec((BLK, F2), lambda i: (i, 0))] * 4,
        out_shape=[jax.ShapeDtypeStruct((NPAD, F2), f32)] * 4,
    )(XH, XG, w12h, bh, w12g, bg, degt)


def _merge(p, degt, sel, add=None, bias=None):
    has_add = add is not None
    has_bias = bias is not None

    def body(*refs):
        i = pl.program_id(0)
        it = iter(refs)
        pr = next(it)[...]
        dg = next(it)[...]
        a0 = next(it)[...] if has_add else 0.0
        a1 = next(it)[...] if has_add else 0.0
        b = next(it)[...] if has_bias else None
        o0 = next(it)
        o1 = next(it)
        dvi, dei, dinv = _scales(dg, i * BLK)
        s = {"dvi": dvi, "dei": dei, "dinv": dinv}[sel]
        v0 = (pr[0, 0] + pr[1, 0] + a0) * s
        v1 = (pr[0, 1] + pr[1, 1] + a1) * s
        if has_bias:
            v0 = (v0 + b[:, :F2]) * s
            v1 = (v1 + b[:, F2:]) * s
        o0[...] = v0
        o1[...] = v1

    in_specs = [
        pl.BlockSpec((NCORE, 2, BLK, F2), lambda i: (0, 0, i, 0)),
        pl.BlockSpec((BLK, 6), lambda i: (i, 0)),
    ]
    args = [p, degt]
    if has_add:
        in_specs.append(pl.BlockSpec((BLK, F2), lambda i: (i, 0)))
        in_specs.append(pl.BlockSpec((BLK, F2), lambda i: (i, 0)))
        args.extend(add)
    if has_bias:
        in_specs.append(pl.BlockSpec((1, F), lambda i: (0, 0)))
        args.append(bias.reshape(1, -1))
    return pl.pallas_call(
        body,
        grid=(NB,),
        in_specs=in_specs,
        out_specs=[pl.BlockSpec((BLK, F2), lambda i: (i, 0))] * 2,
        out_shape=[jax.ShapeDtypeStruct((NPAD, F2), f32)] * 2,
    )(*args)


ABLK = 128
NAB = NPAD // ABLK


def _attn(x, Wa, ba, Wb, bb, Wc, bc):
    def body(xr0, xr1, wa, ba_, wb, bb_, wc, bc_,
             s_o, f_o, mz_o, mr, zr, fr):
        i = pl.program_id(0)

        @pl.when(i == 0)
        def _init():
            mr[...] = jnp.full((1, 1), -1e30, f32)
            zr[...] = jnp.zeros((1, 1), f32)
            fr[...] = jnp.zeros((1, F), f32)

        rid = i * ABLK + lax.broadcasted_iota(jnp.int32, (ABLK, 1), 0)
        mask = rid < N
        xv = jnp.concatenate([xr0[...], xr1[...]], axis=1)
        a = jnp.tanh(jnp.dot(xv, wa[...], precision=HI) + ba_[...])
        bg = jnp.dot(xv, wb[...], precision=HI) + bb_[...]
        bg = 1.0 / (1.0 + jnp.exp(-bg))
        s = jnp.dot(a * bg, wc[...], precision=HI) + bc_[...]
        s = jnp.where(mask, s, -1e30)
        s_o[...] = s
        m_old = mr[0, 0]
        z_old = zr[0, 0]
        m_new = jnp.maximum(m_old, jnp.max(s))
        corr = jnp.exp(m_old - m_new)
        e = jnp.exp(s - m_new)
        z_new = z_old * corr + jnp.sum(e)
        f_new = fr[...] * corr + jnp.sum(e * xv, axis=0, keepdims=True)
        mr[...] = jnp.full((1, 1), m_new, f32)
        zr[...] = jnp.full((1, 1), z_new, f32)
        fr[...] = f_new
        f_o[...] = f_new / z_new
        mz_o[...] = jnp.concatenate(
            [jnp.full((1, 1), m_new, f32), jnp.full((1, 1), z_new, f32)],
            axis=1)

    return pl.pallas_call(
        body,
        grid=(NAB,),
        in_specs=[
            pl.BlockSpec((ABLK, F2), lambda i: (i, 0)),
            pl.BlockSpec((ABLK, F2), lambda i: (i, 0)),
            pl.BlockSpec((F, 256), lambda i: (0, 0)),
            pl.BlockSpec((1, 256), lambda i: (0, 0)),
            pl.BlockSpec((F, 256), lambda i: (0, 0)),
            pl.BlockSpec((1, 256), lambda i: (0, 0)),
            pl.BlockSpec((256, 1), lambda i: (0, 0)),
            pl.BlockSpec((1, 1), lambda i: (0, 0)),
        ],
        out_specs=[
            pl.BlockSpec((ABLK, 1), lambda i: (i, 0)),
            pl.BlockSpec((1, F), lambda i: (0, 0)),
            pl.BlockSpec((1, 2), lambda i: (0, 0)),
        ],
        out_shape=[
            jax.ShapeDtypeStruct((NPAD, 1), f32),
            jax.ShapeDtypeStruct((1, F), f32),
            jax.ShapeDtypeStruct((1, 2), f32),
        ],
        scratch_shapes=[pltpu.VMEM((1, 1), f32), pltpu.VMEM((1, 1), f32),
                        pltpu.VMEM((1, F), f32)],
    )(x[0], x[1], Wa, ba.reshape(1, -1), Wb, bb.reshape(1, -1),
      Wc, bc.reshape(1, -1))


def _head(s2d, mz, feat_h, feat_g, Wo, bo, l1g, l1b, l2g, l2b, Wf, bf):
    def body(s, mzr, fh, fg, wo, bo_, g1, b1, g2, b2, wf, bf_, lo, ws):
        m = mzr[0, 0]
        z = mzr[0, 1]
        ws[...] = jnp.exp(s[...] - m) / z

        def ln(x, gg, bb_):
            mu = jnp.mean(x, axis=-1, keepdims=True)
            va = jnp.mean((x - mu) ** 2, axis=-1, keepdims=True)
            return (x - mu) * lax.rsqrt(va + 1e-5) * gg + bb_

        ha = ln(jnp.dot(fh[...], wo[...], precision=HI) + bo_[...],
                g1[...], b1[...])
        ga = ln(jnp.dot(fg[...], wo[...], precision=HI) + bo_[...],
                g1[...], b1[...])
        xc = ln(jnp.concatenate([ha, ga], axis=1), g2[...], b2[...])
        lo[...] = jnp.dot(xc, wf[...], precision=HI) + bf_[...]

    return pl.pallas_call(
        body,
        out_shape=[
            jax.ShapeDtypeStruct((1, NCLS), f32),
            jax.ShapeDtypeStruct((NPAD // 128, 128), f32),
        ],
    )(s2d, mz, feat_h, feat_g, Wo, bo.reshape(1, -1), l1g.reshape(1, -1),
      l1b.reshape(1, -1), l2g.reshape(1, -1), l2b.reshape(1, -1), Wf,
      bf.reshape(1, -1))


def kernel(X_H, X_G, hg_pairs, g_edge_index, W_h1, b_h1, W_h2, b_h2,
           W_g1, b_g1, W_g2, b_g2, Wa, ba, Wb, bb, Wc, bc, Wo, bo,
           ln1_g, ln1_b, ln2_g, ln2_b, Wf, bf):
    def slab(ix):
        pad = jnp.full((EPAD - E,), PADI, jnp.int32)
        return jnp.concatenate([ix, pad]).reshape(NW, NCHUNK, CHUNK)

    v_s = slab(hg_pairs[0])
    e_s = slab(hg_pairs[1])
    src_s = slab(g_edge_index[0])
    dst_s = slab(g_edge_index[1])
    zrows = jnp.zeros((NPAD - N, D), f32)
    XHp = jnp.concatenate([X_H, zrows], axis=0)
    XGp = jnp.concatenate([X_G, zrows], axis=0)

    zer3 = jnp.zeros((3, NPAD), f32)
    deg = _sc_degrees(v_s, e_s, dst_s, zer3)
    degt = jnp.transpose(deg.reshape(6, NPAD))
    w12h, bh, w12g, bg = _combine_weights(W_h1, W_h2, b_h1, W_g1, W_g2, b_g1)
    mh0, mh1, mg0, mg1 = _project(XHp, XGp, w12h, bh, w12g, bg, degt)
    Mh = (mh0, mh1)
    Mg = (mg0, mg1)

    p = _sc_edge_pass(Mh[0], Mh[1], v_s, e_s)
    q = _sc_edge_pass(Mg[0], Mg[1], src_s, dst_s)
    xe = _merge(p, degt, "dei")
    in2g = _merge(q, degt, "dinv", add=Mg, bias=b_g2)
    p = _sc_edge_pass(xe[0], xe[1], e_s, v_s)
    q = _sc_edge_pass(in2g[0], in2g[1], src_s, dst_s)
    in2 = _merge(p, degt, "dvi", bias=b_h2)
    g = _merge(q, degt, "dinv", add=in2g)
    p = _sc_edge_pass(in2[0], in2[1], v_s, e_s)
    _, feat_g, _ = _attn(g, Wa, ba, Wb, bb, Wc, bc)
    xe2 = _merge(p, degt, "dei")
    p = _sc_edge_pass(xe2[0], xe2[1], e_s, v_s)
    h = _merge(p, degt, "dvi")

    s_h, feat_h, mz_h = _attn(h, Wa, ba, Wb, bb, Wc, bc)
    logits, ws2d = _head(s_h.reshape(NPAD // 128, 128), mz_h, feat_h, feat_g,
                         Wo, bo, ln1_g, ln1_b, ln2_g, ln2_b, Wf, bf)
    return logits, ws2d.reshape(NPAD)[:N]


# speedup vs baseline: 1.1884x; 1.1130x over previous
import functools

import jax
import jax.numpy as jnp
from jax import lax
from jax.experimental import pallas as pl
from jax.experimental.pallas import tpu as pltpu
from jax.experimental.pallas import tpu_sc as plsc

N = 10000
E = 320000
D = 128
F = 64
NCLS = 10
NPAD = 10240
PADI = 10000
NCORE = 2
NSUB = 16
NW = NCORE * NSUB
CHUNK = 128
NCHUNK = 80
EPAD = NW * CHUNK * NCHUNK
RPT = NPAD // NSUB
F2 = F // 2
BLK = 256
NB = NPAD // BLK
HI = lax.Precision.HIGHEST
f32 = jnp.float32

_MESH = plsc.VectorSubcoreMesh(core_axis_name="c", subcore_axis_name="s")


@functools.partial(
    pl.kernel,
    out_type=jax.ShapeDtypeStruct((NCORE, 2, NPAD, F2), f32),
    mesh=_MESH,
    compiler_params=pltpu.CompilerParams(use_tc_tiling_on_sc=False),
    scratch_types=[
        pltpu.VMEM((NCHUNK, CHUNK), jnp.int32),
        pltpu.VMEM((NCHUNK, CHUNK), jnp.int32),
        pltpu.VMEM((4, CHUNK, F2), f32),
        pltpu.VMEM((64, F2), f32),
        pltpu.VMEM_SHARED((NPAD, F2), f32),
        pltpu.VMEM_SHARED((NPAD, F2), f32),
        pltpu.SemaphoreType.DMA,
        pltpu.SemaphoreType.DMA,
    ],
)
def _sc_edge_pass(tab0, tab1, gidx, sidx, out, gi, si, buf, zbuf, src, acc,
                  sem_g, sem_s):
    cid = lax.axis_index("c")
    sid = lax.axis_index("s")
    wid = cid * NSUB + sid
    base = sid * RPT
    pltpu.sync_copy(gidx.at[wid], gi)
    pltpu.sync_copy(sidx.at[wid], si)
    zv = jnp.zeros((16,), f32)
    for r in range(64):
        for c in range(F2 // 16):
            zbuf[r, pl.ds(c * 16, 16)] = zv

    def g_start(j, b):
        pltpu.async_copy(src.at[gi.at[j]], buf.at[b], sem_g)

    def g_wait(j, b):
        pltpu.make_async_copy(src.at[gi.at[j]], buf.at[b], sem_g).wait()

    def s_start(j, b):
        pltpu.async_copy(buf.at[b], acc.at[si.at[j]], sem_s, add=True)

    def s_wait(j, b):
        pltpu.make_async_copy(buf.at[b], acc.at[si.at[j]], sem_s).wait()

    for half, tab in enumerate((tab0, tab1)):
        pltpu.sync_copy(tab.at[pl.ds(base, RPT)], src.at[pl.ds(base, RPT)])

        @pl.loop(0, RPT // 64)
        def _zero(k):
            pltpu.sync_copy(zbuf, acc.at[pl.ds(base + k * 64, 64)])

        plsc.subcore_barrier()
        for b in range(3):
            g_start(b, b)

        g_wait(0, 0)
        s_start(0, 0)
        g_start(3, 3)
        for b in range(1, 4):
            g_wait(b, b)
            s_start(b, b)
            s_wait(b - 1, b - 1)
            g_start(b + 3, (b + 3) % 4)

        @pl.loop(1, NCHUNK // 4 - 1)
        def _steady(k):
            j0 = 4 * k
            for b in range(4):
                j = j0 + b
                g_wait(j, b)
                s_start(j, b)
                s_wait(j - 1, (b - 1) % 4)
                g_start(j + 3, (b + 3) % 4)

        jt = NCHUNK - 4
        g_wait(jt, 0)
        s_start(jt, 0)
        s_wait(jt - 1, 3)
        g_start(NCHUNK - 1, 3)
        for b in range(1, 4):
            g_wait(jt + b, b)
            s_start(jt + b, b)
            s_wait(jt + b - 1, b - 1)
        s_wait(NCHUNK - 1, 3)

        plsc.subcore_barrier()
        pltpu.sync_copy(acc.at[pl.ds(base, RPT)],
                        out.at[cid, half, pl.ds(base, RPT)])


@functools.partial(
    pl.kernel,
    out_type=jax.ShapeDtypeStruct((3, NCORE, NPAD), f32),
    mesh=_MESH,
    compiler_params=pltpu.CompilerParams(use_tc_tiling_on_sc=False),
    scratch_types=[
        pltpu.VMEM((NCHUNK, CHUNK), jnp.int32),
        pltpu.VMEM((CHUNK,), f32),
        pltpu.VMEM_SHARED((NPAD,), f32),
        pltpu.VMEM_SHARED((NPAD,), f32),
        pltpu.VMEM_SHARED((NPAD,), f32),
        pltpu.SemaphoreType.DMA,
    ],
)
def _sc_degrees(vix, eix, dix, zer3, out, iv, ones, t0, t1, t2, sem):
    cid = lax.axis_index("c")
    sid = lax.axis_index("s")
    wid = cid * NSUB + sid
    for c in range(CHUNK // 16):
        ones[pl.ds(c * 16, 16)] = jnp.ones((16,), f32)
    base = sid * RPT
    for k, t in enumerate((t0, t1, t2)):
        pltpu.sync_copy(zer3.at[k, pl.ds(base, RPT)], t.at[pl.ds(base, RPT)])
    plsc.subcore_barrier()
    for slab, t in ((vix, t0), (eix, t1), (dix, t2)):
        pltpu.sync_copy(slab.at[wid], iv)

        @pl.loop(0, NCHUNK // 8)
        def _s(k8, slab=slab, t=t):
            for b in range(8):
                pltpu.async_copy(ones, t.at[iv.at[8 * k8 + b]], sem, add=True)
            pltpu.make_async_copy(slab.at[wid, pl.ds(0, 8)],
                                  iv.at[pl.ds(0, 8)], sem).wait()
    plsc.subcore_barrier()
    for k, t in enumerate((t0, t1, t2)):
        pltpu.sync_copy(t.at[pl.ds(base, RPT)], out.at[k, cid, pl.ds(base, RPT)])


def _scales(dblk, row0):
    rows = dblk.shape[0]
    rid = row0 + lax.broadcasted_iota(jnp.int32, (rows, 1), 0)
    m = (rid < N).astype(f32)
    dv = dblk[:, 0:1] + dblk[:, 1:2]
    de = dblk[:, 2:3] + dblk[:, 3:4]
    dg = dblk[:, 4:5] + dblk[:, 5:6]
    dvi = jnp.where(dv > 0, 1.0, 0.0) * lax.rsqrt(jnp.maximum(dv, 1.0)) * m
    dei = jnp.where(de > 0, 1.0, 0.0) / jnp.maximum(de, 1.0) * m
    dinv = lax.rsqrt(1.0 + dg) * m
    return dvi, dei, dinv


def _combine_weights(W_h1, W_h2, b_h1, W_g1, W_g2, b_g1):
    def body(wh1, wh2, bh1, wg1, wg2, bg1, w12h, bh, w12g, bg):
        w12h[...] = jnp.dot(wh1[...], wh2[...], precision=HI)
        bh[...] = jnp.dot(bh1[...], wh2[...], precision=HI)
        w12g[...] = jnp.dot(wg1[...], wg2[...], precision=HI)
        bg[...] = jnp.dot(bg1[...], wg2[...], precision=HI)

    return pl.pallas_call(
        body,
        out_shape=[
            jax.ShapeDtypeStruct((D, F), f32),
            jax.ShapeDtypeStruct((1, F), f32),
            jax.ShapeDtypeStruct((D, F), f32),
            jax.ShapeDtypeStruct((1, F), f32),
        ],
    )(W_h1, W_h2, b_h1.reshape(1, -1), W_g1, W_g2, b_g1.reshape(1, -1))


def _project(XH, XG, w12h, bh, w12g, bg, degt):
    def body(xh, xg, wh, bh_, wg, bg_, dg, mh0, mh1, mg0, mg1):
        i = pl.program_id(0)
        dvi, _, dinv = _scales(dg[...], i * BLK)
        mh = (jnp.dot(xh[...], wh[...], precision=HI) + bh_[...]) * dvi
        mg = (jnp.dot(xg[...], wg[...], precision=HI) + bg_[...]) * dinv
        mh0[...] = mh[:, :F2]
        mh1[...] = mh[:, F2:]
        mg0[...] = mg[:, :F2]
        mg1[...] = mg[:, F2:]

    return pl.pallas_call(
        body,
        grid=(NB,),
        in_specs=[
            pl.BlockSpec((BLK, D), lambda i: (i, 0)),
            pl.BlockSpec((BLK, D), lambda i: (i, 0)),
            pl.BlockSpec((D, F), lambda i: (0, 0)),
            pl.BlockSpec((1, F), lambda i: (0, 0)),
            pl.BlockSpec((D, F), lambda i: (0, 0)),
            pl.BlockSpec((1, F), lambda i: (0, 0)),
            pl.BlockSpec((BLK, 6), lambda i: (i, 0)),
        ],
        out_specs=[pl.BlockSpec((BLK, F2), lambda i: (i, 0))] * 4,
        out_shape=[jax.ShapeDtypeStruct((NPAD, F2), f32)] * 4,
    )(XH, XG, w12h, bh, w12g, bg, degt)


def _merge(p, degt, sel, add=None, bias=None):
    has_add = add is not None
    has_bias = bias is not None

    def body(*refs):
        i = pl.program_id(0)
        it = iter(refs)
        pr = next(it)[...]
        dg = next(it)[...]
        a0 = next(it)[...] if has_add else 0.0
        a1 = next(it)[...] if has_add else 0.0
        b = next(it)[...] if has_bias else None
        o0 = next(it)
        o1 = next(it)
        dvi, dei, dinv = _scales(dg, i * BLK)
        s = {"dvi": dvi, "dei": dei, "dinv": dinv}[sel]
        v0 = (pr[0, 0] + pr[1, 0] + a0) * s
        v1 = (pr[0, 1] + pr[1, 1] + a1) * s
        if has_bias:
            v0 = (v0 + b[:, :F2]) * s
            v1 = (v1 + b[:, F2:]) * s
        o0[...] = v0
        o1[...] = v1

    in_specs = [
        pl.BlockSpec((NCORE, 2, BLK, F2), lambda i: (0, 0, i, 0)),
        pl.BlockSpec((BLK, 6), lambda i: (i, 0)),
    ]
    args = [p, degt]
    if has_add:
        in_specs.append(pl.BlockSpec((BLK, F2), lambda i: (i, 0)))
        in_specs.append(pl.BlockSpec((BLK, F2), lambda i: (i, 0)))
        args.extend(add)
    if has_bias:
        in_specs.append(pl.BlockSpec((1, F), lambda i: (0, 0)))
        args.append(bias.reshape(1, -1))
    return pl.pallas_call(
        body,
        grid=(NB,),
        in_specs=in_specs,
        out_specs=[pl.BlockSpec((BLK, F2), lambda i: (i, 0))] * 2,
        out_shape=[jax.ShapeDtypeStruct((NPAD, F2), f32)] * 2,
    )(*args)


ABLK = 256
NAB = NPAD // ABLK


def _attn(x, Wa, ba, Wb, bb, Wc, bc):
    def body(xr0, xr1, wa, ba_, wb, bb_, wc, bc_,
             s_o, f_o, mz_o, mr, zr, fr):
        i = pl.program_id(0)

        @pl.when(i == 0)
        def _init():
            mr[...] = jnp.full((1, 1), -1e30, f32)
            zr[...] = jnp.zeros((1, 1), f32)
            fr[...] = jnp.zeros((1, F), f32)

        rid = i * ABLK + lax.broadcasted_iota(jnp.int32, (ABLK, 1), 0)
        mask = rid < N
        xv = jnp.concatenate([xr0[...], xr1[...]], axis=1)
        a = jnp.tanh(jnp.dot(xv, wa[...]) + ba_[...])
        bg = jnp.dot(xv, wb[...]) + bb_[...]
        bg = 1.0 / (1.0 + jnp.exp(-bg))
        s = jnp.dot(a * bg, wc[...]) + bc_[...]
        s = jnp.where(mask, s, -1e30)
        s_o[...] = s
        m_old = mr[0, 0]
        z_old = zr[0, 0]
        m_new = jnp.maximum(m_old, jnp.max(s))
        corr = jnp.exp(m_old - m_new)
        e = jnp.exp(s - m_new)
        z_new = z_old * corr + jnp.sum(e)
        f_new = fr[...] * corr + jnp.sum(e * xv, axis=0, keepdims=True)
        mr[...] = jnp.full((1, 1), m_new, f32)
        zr[...] = jnp.full((1, 1), z_new, f32)
        fr[...] = f_new
        f_o[...] = f_new / z_new
        mz_o[...] = jnp.concatenate(
            [jnp.full((1, 1), m_new, f32), jnp.full((1, 1), z_new, f32)],
            axis=1)

    return pl.pallas_call(
        body,
        grid=(NAB,),
        in_specs=[
            pl.BlockSpec((ABLK, F2), lambda i: (i, 0)),
            pl.BlockSpec((ABLK, F2), lambda i: (i, 0)),
            pl.BlockSpec((F, 256), lambda i: (0, 0)),
            pl.BlockSpec((1, 256), lambda i: (0, 0)),
            pl.BlockSpec((F, 256), lambda i: (0, 0)),
            pl.BlockSpec((1, 256), lambda i: (0, 0)),
            pl.BlockSpec((256, 1), lambda i: (0, 0)),
            pl.BlockSpec((1, 1), lambda i: (0, 0)),
        ],
        out_specs=[
            pl.BlockSpec((ABLK, 1), lambda i: (i, 0)),
            pl.BlockSpec((1, F), lambda i: (0, 0)),
            pl.BlockSpec((1, 2), lambda i: (0, 0)),
        ],
        out_shape=[
            jax.ShapeDtypeStruct((NPAD, 1), f32),
            jax.ShapeDtypeStruct((1, F), f32),
            jax.ShapeDtypeStruct((1, 2), f32),
        ],
        scratch_shapes=[pltpu.VMEM((1, 1), f32), pltpu.VMEM((1, 1), f32),
                        pltpu.VMEM((1, F), f32)],
    )(x[0], x[1], Wa, ba.reshape(1, -1), Wb, bb.reshape(1, -1),
      Wc, bc.reshape(1, -1))


def _head(s2d, mz, feat_h, feat_g, Wo, bo, l1g, l1b, l2g, l2b, Wf, bf):
    def body(s, mzr, fh, fg, wo, bo_, g1, b1, g2, b2, wf, bf_, lo, ws):
        m = mzr[0, 0]
        z = mzr[0, 1]
        ws[...] = jnp.exp(s[...] - m) / z

        def ln(x, gg, bb_):
            mu = jnp.mean(x, axis=-1, keepdims=True)
            va = jnp.mean((x - mu) ** 2, axis=-1, keepdims=True)
            return (x - mu) * lax.rsqrt(va + 1e-5) * gg + bb_

        ha = ln(jnp.dot(fh[...], wo[...], precision=HI) + bo_[...],
                g1[...], b1[...])
        ga = ln(jnp.dot(fg[...], wo[...], precision=HI) + bo_[...],
                g1[...], b1[...])
        xc = ln(jnp.concatenate([ha, ga], axis=1), g2[...], b2[...])
        lo[...] = jnp.dot(xc, wf[...], precision=HI) + bf_[...]

    return pl.pallas_call(
        body,
        out_shape=[
            jax.ShapeDtypeStruct((1, NCLS), f32),
            jax.ShapeDtypeStruct((NPAD // 128, 128), f32),
        ],
    )(s2d, mz, feat_h, feat_g, Wo, bo.reshape(1, -1), l1g.reshape(1, -1),
      l1b.reshape(1, -1), l2g.reshape(1, -1), l2b.reshape(1, -1), Wf,
      bf.reshape(1, -1))


def kernel(X_H, X_G, hg_pairs, g_edge_index, W_h1, b_h1, W_h2, b_h2,
           W_g1, b_g1, W_g2, b_g2, Wa, ba, Wb, bb, Wc, bc, Wo, bo,
           ln1_g, ln1_b, ln2_g, ln2_b, Wf, bf):
    def slab(ix):
        pad = jnp.full((EPAD - E,), PADI, jnp.int32)
        return jnp.concatenate([ix, pad]).reshape(NW, NCHUNK, CHUNK)

    v_s = slab(hg_pairs[0])
    e_s = slab(hg_pairs[1])
    src_s = slab(g_edge_index[0])
    dst_s = slab(g_edge_index[1])
    zrows = jnp.zeros((NPAD - N, D), f32)
    XHp = jnp.concatenate([X_H, zrows], axis=0)
    XGp = jnp.concatenate([X_G, zrows], axis=0)

    zer3 = jnp.zeros((3, NPAD), f32)
    deg = _sc_degrees(v_s, e_s, dst_s, zer3)
    degt = jnp.transpose(deg.reshape(6, NPAD))
    w12h, bh, w12g, bg = _combine_weights(W_h1, W_h2, b_h1, W_g1, W_g2, b_g1)
    mh0, mh1, mg0, mg1 = _project(XHp, XGp, w12h, bh, w12g, bg, degt)
    Mh = (mh0, mh1)
    Mg = (mg0, mg1)

    p = _sc_edge_pass(Mh[0], Mh[1], v_s, e_s)
    q = _sc_edge_pass(Mg[0], Mg[1], src_s, dst_s)
    xe = _merge(p, degt, "dei")
    in2g = _merge(q, degt, "dinv", add=Mg, bias=b_g2)
    p = _sc_edge_pass(xe[0], xe[1], e_s, v_s)
    q = _sc_edge_pass(in2g[0], in2g[1], src_s, dst_s)
    in2 = _merge(p, degt, "dvi", bias=b_h2)
    g = _merge(q, degt, "dinv", add=in2g)
    p = _sc_edge_pass(in2[0], in2[1], v_s, e_s)
    _, feat_g, _ = _attn(g, Wa, ba, Wb, bb, Wc, bc)
    xe2 = _merge(p, degt, "dei")
    p = _sc_edge_pass(xe2[0], xe2[1], e_s, v_s)
    h = _merge(p, degt, "dvi")

    s_h, feat_h, mz_h = _attn(h, Wa, ba, Wb, bb, Wc, bc)
    logits, ws2d = _head(s_h.reshape(NPAD // 128, 128), mz_h, feat_h, feat_g,
                         Wo, bo, ln1_g, ln1_b, ln2_g, ln2_b, Wf, bf)
    return logits, ws2d.reshape(NPAD)[:N]
